# baseline (speedup 1.0000x reference)
import functools

import jax
import jax.numpy as jnp
from jax import lax
from jax.experimental import pallas as pl
from jax.experimental.pallas import tpu as pltpu
from jax.experimental.pallas import tpu_sc as plsc

D = 5000
B = 4 * 2048
NC, NS = 2, 16
NW = NC * NS
B_PER_W = B // NW
CHUNK = 8
LANE_W = 5120
N_CH = B_PER_W // CHUNK
NLH = 1
NSTEP = N_CH * NLH
NBUF = 2


def _sc_gather(idx_flat, C):
    mesh = plsc.VectorSubcoreMesh(core_axis_name="c", subcore_axis_name="s")

    @functools.partial(
        pl.kernel,
        out_type=jax.ShapeDtypeStruct((B, D), jnp.float32),
        mesh=mesh,
        compiler_params=pltpu.CompilerParams(disable_bounds_checks=True),
        scratch_types=[
            pltpu.VMEM((B_PER_W,), jnp.int32),
            [pltpu.VMEM((CHUNK, LANE_W), jnp.float32) for _ in range(NBUF)],
            [pltpu.SemaphoreType.DMA for _ in range(NBUF)],
            [pltpu.SemaphoreType.DMA for _ in range(NBUF)],
        ],
    )
    def k(table_hbm, idx_hbm, out_hbm, idx_v, bufs, gsems, wsems):
        wid = lax.axis_index("s") * NC + lax.axis_index("c")
        base = wid * B_PER_W
        pltpu.sync_copy(idx_hbm.at[pl.ds(base, B_PER_W)], idx_v)

        def gdesc(s, p):
            c = s // NLH
            h = s % NLH
            lane = pl.multiple_of(h * LANE_W, 128)
            return pltpu.make_async_copy(
                table_hbm.at[idx_v.at[pl.ds(c * CHUNK, CHUNK)], pl.ds(lane, LANE_W)],
                bufs[p],
                gsems[p],
            )

        def wdesc(s, p):
            c = s // NLH
            h = s % NLH
            lane = pl.multiple_of(h * LANE_W, 128)
            return pltpu.make_async_copy(
                bufs[p],
                out_hbm.at[pl.ds(base + c * CHUNK, CHUNK), pl.ds(lane, LANE_W)],
                wsems[p],
            )

        zero = wid * 0
        for p in range(NBUF):
            gdesc(zero + p, p).start()

        @pl.loop(0, NSTEP - NBUF, step=NBUF)
        def _(s):
            for p in range(NBUF):
                gdesc(s + p, p).wait()
                wdesc(s + p, p).start()
            for p in range(NBUF):
                wdesc(s + p, p).wait()
                gdesc(s + NBUF + p, p).start()

        for p in range(NBUF):
            gdesc(zero + NSTEP - NBUF + p, p).wait()
            wdesc(zero + NSTEP - NBUF + p, p).start()
        for p in range(NBUF):
            wdesc(zero + NSTEP - NBUF + p, p).wait()

    return k(C, idx_flat)


def kernel(x, C):
    idx = x.reshape(-1).astype(jnp.int32)
    out = _sc_gather(idx, C)
    return out.reshape(x.shape[0], x.shape[1], D)


# speedup vs baseline: 1.0138x; 1.0138x over previous
import functools

import jax
import jax.numpy as jnp
from jax import lax
from jax.experimental import pallas as pl
from jax.experimental.pallas import tpu as pltpu
from jax.experimental.pallas import tpu_sc as plsc

D = 5000
B = 4 * 2048
NC, NS = 2, 16
NW = NC * NS
B_PER_W = B // NW
CHUNK = 8
LANE_W = 1280
N_CH = B_PER_W // CHUNK
NLH = 4
NSTEP = N_CH * NLH
NBUF = 8


def _sc_gather(idx_flat, C):
    mesh = plsc.VectorSubcoreMesh(core_axis_name="c", subcore_axis_name="s")

    @functools.partial(
        pl.kernel,
        out_type=jax.ShapeDtypeStruct((B, D), jnp.float32),
        mesh=mesh,
        compiler_params=pltpu.CompilerParams(disable_bounds_checks=True),
        scratch_types=[
            pltpu.VMEM((B_PER_W,), jnp.int32),
            [pltpu.VMEM((CHUNK, LANE_W), jnp.float32) for _ in range(NBUF)],
            [pltpu.SemaphoreType.DMA for _ in range(NBUF)],
            [pltpu.SemaphoreType.DMA for _ in range(NBUF)],
        ],
    )
    def k(table_hbm, idx_hbm, out_hbm, idx_v, bufs, gsems, wsems):
        wid = lax.axis_index("s") * NC + lax.axis_index("c")
        base = wid * B_PER_W
        pltpu.sync_copy(idx_hbm.at[pl.ds(base, B_PER_W)], idx_v)

        def gdesc(s, p):
            c = s // NLH
            h = s % NLH
            lane = pl.multiple_of(h * LANE_W, 128)
            return pltpu.make_async_copy(
                table_hbm.at[idx_v.at[pl.ds(c * CHUNK, CHUNK)], pl.ds(lane, LANE_W)],
                bufs[p],
                gsems[p],
            )

        def wdesc(s, p):
            c = s // NLH
            h = s % NLH
            lane = pl.multiple_of(h * LANE_W, 128)
            return pltpu.make_async_copy(
                bufs[p],
                out_hbm.at[pl.ds(base + c * CHUNK, CHUNK), pl.ds(lane, LANE_W)],
                wsems[p],
            )

        zero = wid * 0
        for p in range(NBUF):
            gdesc(zero + p, p).start()

        @pl.loop(0, NSTEP - NBUF, step=NBUF)
        def _(s):
            for p in range(NBUF):
                gdesc(s + p, p).wait()
                wdesc(s + p, p).start()
            for p in range(NBUF):
                wdesc(s + p, p).wait()
                gdesc(s + NBUF + p, p).start()

        for p in range(NBUF):
            gdesc(zero + NSTEP - NBUF + p, p).wait()
            wdesc(zero + NSTEP - NBUF + p, p).start()
        for p in range(NBUF):
            wdesc(zero + NSTEP - NBUF + p, p).wait()

    return k(C, idx_flat)


def kernel(x, C):
    idx = x.reshape(-1).astype(jnp.int32)
    out = _sc_gather(idx, C)
    return out.reshape(x.shape[0], x.shape[1], D)
